# baseline (speedup 1.0000x reference)
import functools

import jax
import jax.numpy as jnp
import numpy as np
from jax import lax
from jax.experimental import pallas as pl
from jax.experimental.pallas import tpu as pltpu
from jax.experimental.pallas import tpu_sc as plsc

B, T, S, F = 1024, 20, 2, 26
V, D = 100000, 32
N = B * T * S
M = N * F
NW = 32
PER_W = M // NW
GROUP = 128
NG = 13
BATCH = GROUP * NG
NB = PER_W // BATCH
L = 16


def _body(xf, tab, out, idx_a, idx_b, offb, rows_a, rows_b,
          gsem_a, gsem_b, osem_a, osem_b):
    wid = lax.axis_index("s") * 2 + lax.axis_index("c")
    wbase = wid * PER_W

    @pl.loop(0, BATCH // L)
    def _off(k):
        pos = k * L + lax.iota(jnp.int32, L)
        offb[pl.ds(k * L, L)] = (pos % F) * V

    def load_compute(b, idxb):
        pltpu.sync_copy(xf.at[pl.ds(wbase + b * BATCH, BATCH)], idxb)

        @pl.loop(0, BATCH // L)
        def _add(k):
            idxb[pl.ds(k * L, L)] = idxb[pl.ds(k * L, L)] + offb[pl.ds(k * L, L)]

    def fire_gathers(idxb, rows, gsem):
        for g in range(NG):
            pltpu.async_copy(
                tab.at[idxb.at[pl.ds(g * GROUP, GROUP)]],
                rows.at[pl.ds(g * GROUP, GROUP)],
                gsem,
            )

    def wait_gathers(rows, gsem):
        pltpu.make_async_copy(out.at[pl.ds(0, BATCH)], rows, gsem).wait()

    def fire_out(b, rows, osem):
        pltpu.async_copy(rows, out.at[pl.ds(wbase + b * BATCH, BATCH)], osem)

    def wait_out(rows, osem):
        pltpu.make_async_copy(rows, out.at[pl.ds(0, BATCH)], osem).wait()

    load_compute(0, idx_a)
    fire_gathers(idx_a, rows_a, gsem_a)
    load_compute(1, idx_b)
    fire_gathers(idx_b, rows_b, gsem_b)

    @pl.loop(0, (NB - 2) // 2)
    def _iter(i):
        b = 2 * i
        wait_gathers(rows_a, gsem_a)
        fire_out(b, rows_a, osem_a)
        load_compute(b + 2, idx_a)
        wait_out(rows_a, osem_a)
        fire_gathers(idx_a, rows_a, gsem_a)

        wait_gathers(rows_b, gsem_b)
        fire_out(b + 1, rows_b, osem_b)
        load_compute(b + 3, idx_b)
        wait_out(rows_b, osem_b)
        fire_gathers(idx_b, rows_b, gsem_b)

    wait_gathers(rows_a, gsem_a)
    fire_out(NB - 2, rows_a, osem_a)
    wait_gathers(rows_b, gsem_b)
    fire_out(NB - 1, rows_b, osem_b)
    wait_out(rows_a, osem_a)
    wait_out(rows_b, osem_b)


@jax.jit
def _embed(xf, tab):
    mesh = plsc.VectorSubcoreMesh(core_axis_name="c", subcore_axis_name="s")
    return pl.kernel(
        _body,
        out_type=jax.ShapeDtypeStruct((M, D), jnp.float32),
        mesh=mesh,
        scratch_types=[
            pltpu.VMEM((BATCH,), jnp.int32),
            pltpu.VMEM((BATCH,), jnp.int32),
            pltpu.VMEM((BATCH,), jnp.int32),
            pltpu.VMEM((BATCH, D), jnp.float32),
            pltpu.VMEM((BATCH, D), jnp.float32),
            pltpu.SemaphoreType.DMA,
            pltpu.SemaphoreType.DMA,
            pltpu.SemaphoreType.DMA,
            pltpu.SemaphoreType.DMA,
        ],
        compiler_params=pltpu.CompilerParams(use_tc_tiling_on_sc=False),
    )(xf, tab)


_VB = 512
_RB = _VB // 4

def _mk_sel():
    s = np.zeros((4, _RB, _VB), np.float32)
    for q in range(4):
        for r in range(_RB):
            s[q, r, 4 * r + q] = 1.0
    return jnp.asarray(s)

def _mk_plc():
    p = np.zeros((4, D, 4 * D), np.float32)
    for q in range(4):
        for d in range(D):
            p[q, d, D * q + d] = 1.0
    return jnp.asarray(p)


def _fmt_body(sel_ref, plc_ref, in_ref, out_ref):
    x = in_ref[0]
    acc = jnp.zeros((_RB, 4 * D), jnp.float32)
    for q in range(4):
        xt_p = lax.dot_general(x, plc_ref[q], (((0,), (0,)), ((), ())),
                               precision=lax.Precision.HIGHEST,
                               preferred_element_type=jnp.float32)
        acc = acc + lax.dot_general(sel_ref[q], xt_p,
                                    (((1,), (0,)), ((), ())),
                                    precision=lax.Precision.HIGHEST,
                                    preferred_element_type=jnp.float32)
    out_ref[0] = acc


_N_VBLK = -(-V // _VB)


@jax.jit
def _format_table(tt):
    return pl.pallas_call(
        _fmt_body,
        grid=(F, _N_VBLK),
        in_specs=[
            pl.BlockSpec((4, _RB, _VB), lambda f, c: (0, 0, 0)),
            pl.BlockSpec((4, D, 4 * D), lambda f, c: (0, 0, 0)),
            pl.BlockSpec((1, D, _VB), lambda f, c: (f, 0, c)),
        ],
        out_specs=pl.BlockSpec((1, _RB, 4 * D), lambda f, c: (f, c, 0)),
        out_shape=jax.ShapeDtypeStruct((F, V // 4, 4 * D), jnp.float32),
    )(_mk_sel(), _mk_plc(), tt)


def kernel(x, tables):
    tt = tables.transpose(0, 2, 1)
    tabT = _format_table(tt)
    tab = tabT.reshape(F * V, D)
    xf = x.reshape(M)
    out = _embed(xf, tab)
    return out.reshape(B, T, S, F * D)


# speedup vs baseline: 2.7462x; 2.7462x over previous
import functools

import jax
import jax.numpy as jnp
import numpy as np
from jax import lax
from jax.experimental import pallas as pl
from jax.experimental.pallas import tpu as pltpu
from jax.experimental.pallas import tpu_sc as plsc

B, T, S, F = 1024, 20, 2, 26
V, D = 100000, 32
N = B * T * S
M = N * F
NW = 32
PER_W = M // NW
GROUP = 128
NG = 13
BATCH = GROUP * NG
NB = PER_W // BATCH
L = 16


def _body(xf, tab, out, idx_a, idx_b, offb, rows_a, rows_b,
          gsem_a, gsem_b, osem_a, osem_b):
    wid = lax.axis_index("s") * 2 + lax.axis_index("c")
    wbase = wid * PER_W

    @pl.loop(0, BATCH // L)
    def _off(k):
        pos = k * L + lax.iota(jnp.int32, L)
        offb[pl.ds(k * L, L)] = (pos % F) * V

    def load_compute(b, idxb):
        pltpu.sync_copy(xf.at[pl.ds(wbase + b * BATCH, BATCH)], idxb)

        @pl.loop(0, BATCH // L)
        def _add(k):
            idxb[pl.ds(k * L, L)] = idxb[pl.ds(k * L, L)] + offb[pl.ds(k * L, L)]

    def fire_gathers(idxb, rows, gsem):
        for g in range(NG):
            pltpu.async_copy(
                tab.at[idxb.at[pl.ds(g * GROUP, GROUP)]],
                rows.at[pl.ds(g * GROUP, GROUP)],
                gsem,
            )

    def wait_gathers(rows, gsem):
        pltpu.make_async_copy(out.at[pl.ds(0, BATCH)], rows, gsem).wait()

    def fire_out(b, rows, osem):
        pltpu.async_copy(rows, out.at[pl.ds(wbase + b * BATCH, BATCH)], osem)

    def wait_out(rows, osem):
        pltpu.make_async_copy(rows, out.at[pl.ds(0, BATCH)], osem).wait()

    load_compute(0, idx_a)
    fire_gathers(idx_a, rows_a, gsem_a)
    load_compute(1, idx_b)
    fire_gathers(idx_b, rows_b, gsem_b)

    @pl.loop(0, (NB - 2) // 2)
    def _iter(i):
        b = 2 * i
        wait_gathers(rows_a, gsem_a)
        fire_out(b, rows_a, osem_a)
        load_compute(b + 2, idx_a)
        wait_out(rows_a, osem_a)
        fire_gathers(idx_a, rows_a, gsem_a)

        wait_gathers(rows_b, gsem_b)
        fire_out(b + 1, rows_b, osem_b)
        load_compute(b + 3, idx_b)
        wait_out(rows_b, osem_b)
        fire_gathers(idx_b, rows_b, gsem_b)

    wait_gathers(rows_a, gsem_a)
    fire_out(NB - 2, rows_a, osem_a)
    wait_gathers(rows_b, gsem_b)
    fire_out(NB - 1, rows_b, osem_b)
    wait_out(rows_a, osem_a)
    wait_out(rows_b, osem_b)


@jax.jit
def _embed(xf, tab):
    mesh = plsc.VectorSubcoreMesh(core_axis_name="c", subcore_axis_name="s")
    return pl.kernel(
        _body,
        out_type=jax.ShapeDtypeStruct((M, D), jnp.float32),
        mesh=mesh,
        scratch_types=[
            pltpu.VMEM((BATCH,), jnp.int32),
            pltpu.VMEM((BATCH,), jnp.int32),
            pltpu.VMEM((BATCH,), jnp.int32),
            pltpu.VMEM((BATCH, D), jnp.float32),
            pltpu.VMEM((BATCH, D), jnp.float32),
            pltpu.SemaphoreType.DMA,
            pltpu.SemaphoreType.DMA,
            pltpu.SemaphoreType.DMA,
            pltpu.SemaphoreType.DMA,
        ],
        compiler_params=pltpu.CompilerParams(use_tc_tiling_on_sc=False),
    )(xf, tab)


_VB = 512
_RB = _VB // 4

def _mk_sel():
    s = np.zeros((4, _RB, _VB), np.float32)
    for q in range(4):
        for r in range(_RB):
            s[q, r, 4 * r + q] = 1.0
    return jnp.asarray(s)

def _mk_plc():
    p = np.zeros((4, D, 4 * D), np.float32)
    for q in range(4):
        for d in range(D):
            p[q, d, D * q + d] = 1.0
    return jnp.asarray(p)


def _fmt_body(sel_ref, plc_ref, in_ref, out_ref):
    x = in_ref[0]
    acc = jnp.zeros((_RB, 4 * D), jnp.float32)
    for q in range(4):
        xt_p = lax.dot_general(x, plc_ref[q], (((0,), (0,)), ((), ())),
                               preferred_element_type=jnp.float32)
        acc = acc + lax.dot_general(sel_ref[q], xt_p,
                                    (((1,), (0,)), ((), ())),
                                    preferred_element_type=jnp.float32)
    out_ref[0] = acc


_N_VBLK = -(-V // _VB)


@jax.jit
def _format_table(tt):
    return pl.pallas_call(
        _fmt_body,
        grid=(F, _N_VBLK),
        in_specs=[
            pl.BlockSpec((4, _RB, _VB), lambda f, c: (0, 0, 0)),
            pl.BlockSpec((4, D, 4 * D), lambda f, c: (0, 0, 0)),
            pl.BlockSpec((1, D, _VB), lambda f, c: (f, 0, c)),
        ],
        out_specs=pl.BlockSpec((1, _RB, 4 * D), lambda f, c: (f, c, 0)),
        out_shape=jax.ShapeDtypeStruct((F, V // 4, 4 * D), jnp.float32),
    )(_mk_sel(), _mk_plc(), tt)


def kernel(x, tables):
    tt = tables.transpose(0, 2, 1)
    tabT = _format_table(tt)
    tab = tabT.reshape(F * V, D)
    xf = x.reshape(M)
    out = _embed(xf, tab)
    return out.reshape(B, T, S, F * D)


# speedup vs baseline: 6.4831x; 2.3607x over previous
import functools

import jax
import jax.numpy as jnp
from jax import lax
from jax.experimental import pallas as pl
from jax.experimental.pallas import tpu as pltpu
from jax.experimental.pallas import tpu_sc as plsc

B, T, S, F = 1024, 20, 2, 26
V, D = 100000, 32
N = B * T * S
M = N * F
Z = T * S * F
NW = 32
NSLOT = (Z + NW - 1) // NW
L = 16


def _body(xf2, tab, out6, idxb, gidx_a, gidx_b, rows_a, rows_b, slab,
          gsem_a, gsem_b, osem):
    w = lax.axis_index("s") * 2 + lax.axis_index("c")
    iota16 = lax.iota(jnp.int32, L)

    def unit_tsf(u):
        t = u // (S * F)
        r = u % (S * F)
        return t, r // F, r % F

    def stage1(u, gidx, rows, gsem):
        _, _, f = unit_tsf(u)
        c8 = (u // 8) * 8
        col = u % 8
        pltpu.sync_copy(xf2.at[pl.ds(0, B), pl.ds(c8, 8)], idxb)
        fv = f * V

        @pl.loop(0, B // L)
        def _x(k):
            rowi = k * L + iota16
            coli = jnp.full((L,), col, jnp.int32)
            gidx[pl.ds(k * L, L)] = plsc.load_gather(idxb, [rowi, coli]) + fv

        for g in range(8):
            pltpu.async_copy(
                tab.at[gidx.at[pl.ds(g * 128, 128)]],
                rows.at[pl.ds(g * 128, 128)],
                gsem,
            )

    def wait_gathers(rows, gsem):
        pltpu.make_async_copy(tab.at[pl.ds(0, B)], rows, gsem).wait()

    def slab_dst(u):
        t, s, f = unit_tsf(u)
        return out6.at[t, s, pl.ds(4 * f, 4)]

    def wait_slab(u):
        pltpu.make_async_copy(slab, slab_dst(u), osem).wait()

    def stage2(u, rows, gsem):
        wait_gathers(rows, gsem)
        wait_slab(u)

        @pl.loop(0, D)
        def _d(d):
            coli = jnp.full((L,), d, jnp.int32)
            dd = d // 8
            dm = d % 8

            @pl.loop(0, 8)
            def _t(bc):
                for k in range(8):
                    rowi = bc * 128 + (k * L + iota16)
                    vals = plsc.load_gather(rows, [rowi, coli])
                    slab[dd, bc, dm, pl.ds(k * L, L)] = vals
        pltpu.async_copy(slab, slab_dst(u), osem)

    def valid(j):
        return w + NW * j < Z

    pltpu.async_copy(slab, slab_dst(w), osem)

    stage1(w, gidx_a, rows_a, gsem_a)
    stage1(w + NW, gidx_b, rows_b, gsem_b)

    @pl.loop(0, (NSLOT - 1) // 2)
    def _iter(i):
        ja = 2 * i
        jb = 2 * i + 1
        ua = w + NW * ja
        ub = w + NW * jb
        stage2(ua, rows_a, gsem_a)

        @pl.when(valid(ja + 2))
        def _s1a():
            stage1(w + NW * (ja + 2), gidx_a, rows_a, gsem_a)

        stage2(ub, rows_b, gsem_b)

        @pl.when(valid(jb + 2))
        def _s1b():
            stage1(w + NW * (jb + 2), gidx_b, rows_b, gsem_b)

    @pl.when(valid(NSLOT - 1))
    def _tail():
        stage2(w + NW * (NSLOT - 1), rows_a, gsem_a)

    wait_slab(w)


@jax.jit
def _embed(xf2, tab):
    mesh = plsc.VectorSubcoreMesh(core_axis_name="c", subcore_axis_name="s")
    return pl.kernel(
        _body,
        out_type=jax.ShapeDtypeStruct((T, S, 104, 8, 8, 128), jnp.float32),
        mesh=mesh,
        scratch_types=[
            pltpu.VMEM((B, 8), jnp.int32),
            pltpu.VMEM((B,), jnp.int32),
            pltpu.VMEM((B,), jnp.int32),
            pltpu.VMEM((B, D), jnp.float32),
            pltpu.VMEM((B, D), jnp.float32),
            pltpu.VMEM((4, 8, 8, 128), jnp.float32),
            pltpu.SemaphoreType.DMA,
            pltpu.SemaphoreType.DMA,
            pltpu.SemaphoreType.DMA,
        ],
        compiler_params=pltpu.CompilerParams(
            use_tc_tiling_on_sc=False, needs_layout_passes=False),
    )(xf2, tab)


def kernel(x, tables):
    xf2 = x.reshape(B, T * S * F)
    tab = tables.reshape(F * V, D)
    out6 = _embed(xf2, tab)
    return out6.transpose(3, 5, 0, 1, 2, 4).reshape(B, T, S, F * D)


# speedup vs baseline: 7.5176x; 1.1596x over previous
import functools

import jax
import jax.numpy as jnp
from jax import lax
from jax.experimental import pallas as pl
from jax.experimental.pallas import tpu as pltpu
from jax.experimental.pallas import tpu_sc as plsc

B, T, S, F = 1024, 20, 2, 26
V, D = 100000, 32
N = B * T * S
M = N * F
NW = 32
PER_W = M // NW
GROUP = 128
NG = 13
BATCH = GROUP * NG
NB = PER_W // BATCH
L = 16


def _body(xf, tab, out, idx_a, idx_b, offb, rows_a, rows_b,
          gsem_a, gsem_b, osem_a, osem_b):
    wid = lax.axis_index("s") * 2 + lax.axis_index("c")
    wbase = wid * PER_W

    @pl.loop(0, BATCH // L)
    def _off(k):
        pos = k * L + lax.iota(jnp.int32, L)
        offb[pl.ds(k * L, L)] = (pos % F) * V

    def load_compute(b, idxb):
        pltpu.sync_copy(xf.at[pl.ds(wbase + b * BATCH, BATCH)], idxb)

        @pl.loop(0, BATCH // L)
        def _add(k):
            idxb[pl.ds(k * L, L)] = idxb[pl.ds(k * L, L)] + offb[pl.ds(k * L, L)]

    def fire_gathers(idxb, rows, gsem):
        for g in range(NG):
            pltpu.async_copy(
                tab.at[idxb.at[pl.ds(g * GROUP, GROUP)]],
                rows.at[pl.ds(g * GROUP, GROUP)],
                gsem,
            )

    def wait_gathers(rows, gsem):
        pltpu.make_async_copy(out.at[pl.ds(0, BATCH)], rows, gsem).wait()

    def fire_out(b, rows, osem):
        pltpu.async_copy(rows, out.at[pl.ds(wbase + b * BATCH, BATCH)], osem)

    def wait_out(rows, osem):
        pltpu.make_async_copy(rows, out.at[pl.ds(0, BATCH)], osem).wait()

    load_compute(0, idx_a)
    fire_gathers(idx_a, rows_a, gsem_a)
    load_compute(1, idx_b)
    fire_gathers(idx_b, rows_b, gsem_b)

    @pl.loop(0, (NB - 2) // 2)
    def _iter(i):
        b = 2 * i
        wait_gathers(rows_a, gsem_a)
        fire_out(b, rows_a, osem_a)
        load_compute(b + 2, idx_a)
        wait_out(rows_a, osem_a)
        fire_gathers(idx_a, rows_a, gsem_a)

        wait_gathers(rows_b, gsem_b)
        fire_out(b + 1, rows_b, osem_b)
        load_compute(b + 3, idx_b)
        wait_out(rows_b, osem_b)
        fire_gathers(idx_b, rows_b, gsem_b)

    wait_gathers(rows_a, gsem_a)
    fire_out(NB - 2, rows_a, osem_a)
    wait_gathers(rows_b, gsem_b)
    fire_out(NB - 1, rows_b, osem_b)
    wait_out(rows_a, osem_a)
    wait_out(rows_b, osem_b)


@jax.jit
def _embed(xf, tab):
    mesh = plsc.VectorSubcoreMesh(core_axis_name="c", subcore_axis_name="s")
    return pl.kernel(
        _body,
        out_type=jax.ShapeDtypeStruct((M, D), jnp.float32),
        mesh=mesh,
        scratch_types=[
            pltpu.VMEM((BATCH,), jnp.int32),
            pltpu.VMEM((BATCH,), jnp.int32),
            pltpu.VMEM((BATCH,), jnp.int32),
            pltpu.VMEM((BATCH, D), jnp.float32),
            pltpu.VMEM((BATCH, D), jnp.float32),
            pltpu.SemaphoreType.DMA,
            pltpu.SemaphoreType.DMA,
            pltpu.SemaphoreType.DMA,
            pltpu.SemaphoreType.DMA,
        ],
        compiler_params=pltpu.CompilerParams(use_tc_tiling_on_sc=False),
    )(xf, tab)


def kernel(x, tables):
    xf = x.reshape(M)
    tab = tables.reshape(F * V, D)
    out = _embed(xf, tab)
    return out.reshape(B, T, S, F * D)


# speedup vs baseline: 7.5196x; 1.0003x over previous
import jax
import jax.numpy as jnp
from jax import lax
from jax.experimental import pallas as pl
from jax.experimental.pallas import tpu as pltpu
from jax.experimental.pallas import tpu_sc as plsc

B, T, S, F = 1024, 20, 2, 26
V, D = 100000, 32
N = B * T * S
M = N * F
NW = 32
PER_W = M // NW
GROUP = 128
NG = 13
BATCH = GROUP * NG
NB = PER_W // BATCH
L = 16


def _body(xf, tab, out, idx_a, idx_b, offb, rows_a, rows_b,
          gsem_a, gsem_b, osem_a, osem_b):
    wid = lax.axis_index("s") * 2 + lax.axis_index("c")
    wbase = wid * PER_W

    @pl.loop(0, BATCH // L)
    def _off(k):
        pos = k * L + lax.iota(jnp.int32, L)
        offb[pl.ds(k * L, L)] = (pos % F) * V

    def load_compute(b, idxb):
        pltpu.sync_copy(xf.at[pl.ds(wbase + b * BATCH, BATCH)], idxb)

        @pl.loop(0, BATCH // L)
        def _add(k):
            idxb[pl.ds(k * L, L)] = idxb[pl.ds(k * L, L)] + offb[pl.ds(k * L, L)]

    def fire_gathers(idxb, rows, gsem):
        for g in range(NG):
            pltpu.async_copy(
                tab.at[idxb.at[pl.ds(g * GROUP, GROUP)]],
                rows.at[pl.ds(g * GROUP, GROUP)],
                gsem,
            )

    def wait_gathers(rows, gsem):
        pltpu.make_async_copy(out.at[pl.ds(0, BATCH)], rows, gsem).wait()

    def fire_out(b, rows, osem):
        pltpu.async_copy(rows, out.at[pl.ds(wbase + b * BATCH, BATCH)], osem)

    def wait_out(rows, osem):
        pltpu.make_async_copy(rows, out.at[pl.ds(0, BATCH)], osem).wait()

    load_compute(0, idx_a)
    fire_gathers(idx_a, rows_a, gsem_a)
    load_compute(1, idx_b)
    fire_gathers(idx_b, rows_b, gsem_b)

    @pl.loop(0, (NB - 2) // 2)
    def _iter(i):
        b = 2 * i
        wait_gathers(rows_a, gsem_a)
        fire_out(b, rows_a, osem_a)
        load_compute(b + 2, idx_a)
        wait_out(rows_a, osem_a)
        fire_gathers(idx_a, rows_a, gsem_a)

        wait_gathers(rows_b, gsem_b)
        fire_out(b + 1, rows_b, osem_b)
        load_compute(b + 3, idx_b)
        wait_out(rows_b, osem_b)
        fire_gathers(idx_b, rows_b, gsem_b)

    wait_gathers(rows_a, gsem_a)
    fire_out(NB - 2, rows_a, osem_a)
    wait_gathers(rows_b, gsem_b)
    fire_out(NB - 1, rows_b, osem_b)
    wait_out(rows_a, osem_a)
    wait_out(rows_b, osem_b)


@jax.jit
def _embed(xf, tab):
    mesh = plsc.VectorSubcoreMesh(core_axis_name="c", subcore_axis_name="s")
    return pl.kernel(
        _body,
        out_type=jax.ShapeDtypeStruct((M, D), jnp.float32),
        mesh=mesh,
        scratch_types=[
            pltpu.VMEM((BATCH,), jnp.int32),
            pltpu.VMEM((BATCH,), jnp.int32),
            pltpu.VMEM((BATCH,), jnp.int32),
            pltpu.VMEM((BATCH, D), jnp.float32),
            pltpu.VMEM((BATCH, D), jnp.float32),
            pltpu.SemaphoreType.DMA,
            pltpu.SemaphoreType.DMA,
            pltpu.SemaphoreType.DMA,
            pltpu.SemaphoreType.DMA,
        ],
        compiler_params=pltpu.CompilerParams(use_tc_tiling_on_sc=False),
    )(xf, tab)


def kernel(x, tables):
    xf = x.reshape(M)
    tab = tables.reshape(F * V, D)
    out = _embed(xf, tab)
    return out.reshape(B, T, S, F * D)
